# baseline (speedup 1.0000x reference)
import functools

import jax
import jax.numpy as jnp
from jax import lax
from jax.experimental import pallas as pl
from jax.experimental.pallas import tpu as pltpu
from jax.experimental.pallas import tpu_sc as plsc

N = 10000
E = 320000
F = 128
H = 256
NP = 10240
NC = 2
NS = 16
ROWS_PER_TILE = NP // NS
CE = 100
CHUNKS_B = (E // NC) // NS // CE
CHUNKS_A = E // NS // CE
XROWS = 80
W16 = 16
KDEG = 4


def _vec_fill(ref, n16, value):
    v = jnp.full((16,), value, dtype=ref.dtype)

    def body(i, _):
        ref[pl.ds(i * 16, 16)] = v
        return 0

    lax.fori_loop(0, n16, body, 0)


def _vec_fill2d(ref, value):
    rows, cols = ref.shape
    v = jnp.full((16,), value, dtype=ref.dtype)

    def body(r, _):
        for u in range(cols // 16):
            ref[r, pl.ds(u * 16, 16)] = v
        return 0

    lax.fori_loop(0, rows, body, 0)


def _scale_rows(buf, dis_ref, dis_base, nrows, negate):
    def group(g, _):
        dvec = dis_ref[pl.ds(dis_base + g * 16, 16)]
        if negate:
            dvec = 0.0 - dvec
        for r16 in range(16):
            s = dvec[r16]
            row = g * 16 + r16
            for u in range(F // 16):
                buf[row, pl.ds(u * 16, 16)] = buf[row, pl.ds(u * 16, 16)] * s
        return 0

    lax.fori_loop(0, nrows // 16, group, 0)


def _fast_rsqrt(d):
    i = lax.bitcast_convert_type(d, jnp.int32)
    i = jnp.int32(0x5F3759DF) - (i >> 1)
    y = lax.bitcast_convert_type(i, jnp.float32)
    for _ in range(3):
        y = y * (1.5 - 0.5 * d * y * y)
    return y


def _deg_xs_body(col3, x_hbm, xs_out, dis_out, col_all, ones_buf, deg2,
                 dis_buf, xbuf, deg_sp, sem):
    c = lax.axis_index("c")
    t = lax.axis_index("s")

    _vec_fill(deg2, ROWS_PER_TILE // 16, 0.0)
    deg_slice = deg_sp.at[pl.ds(t * ROWS_PER_TILE, ROWS_PER_TILE)]
    pltpu.sync_copy(deg2, deg_slice)
    _vec_fill(ones_buf, CE // 16, 1.0)
    pltpu.sync_copy(col3.at[t], col_all)
    plsc.subcore_barrier()

    def deg_step(i, _):
        pltpu.sync_copy(ones_buf, deg_sp.at[col_all.at[i]], add=True)
        return 0

    lax.fori_loop(0, CHUNKS_A, deg_step, 0)
    plsc.subcore_barrier()

    pltpu.sync_copy(deg_slice, deg2)

    def dis_step(j, _):
        deg = deg2[pl.ds(j * 16, 16)]
        y = _fast_rsqrt(jnp.maximum(deg, 1.0))
        dis_buf[pl.ds(j * 16, 16)] = jnp.where(deg > 0.0, y, 0.0)
        return 0

    lax.fori_loop(0, ROWS_PER_TILE // 16, dis_step, 0)

    @pl.when(c == 0)
    def _():
        pltpu.sync_copy(dis_buf, dis_out.at[pl.ds(t * ROWS_PER_TILE,
                                                  ROWS_PER_TILE)])

    row0 = t * ROWS_PER_TILE + c * (ROWS_PER_TILE // 2)
    nch = jnp.minimum(jnp.maximum(N - row0, 0), ROWS_PER_TILE // 2) // XROWS

    def x_chunk(k, _):
        r0 = row0 + k * XROWS
        pltpu.sync_copy(x_hbm.at[pl.ds(r0, XROWS)], xbuf)
        _scale_rows(xbuf, dis_buf, r0 - t * ROWS_PER_TILE, XROWS,
                    negate=False)
        pltpu.sync_copy(xbuf, xs_out.at[pl.ds(r0, XROWS)])
        return 0

    lax.fori_loop(0, nch, x_chunk, 0)


def _edge_body(row3, col3, xs_hbm, dis_hbm, accs_out, row_all, col_all,
               buf0, buf1, dis_buf, acc_sp, sem0, sem1, ssem):
    c = lax.axis_index("c")
    t = lax.axis_index("s")

    _vec_fill2d(buf0, 0.0)
    zsrc = buf0.at[pl.ds(0, 80)]

    def zero_step(k, _):
        pltpu.sync_copy(zsrc, acc_sp.at[pl.ds(t * ROWS_PER_TILE + k * 80, 80)])
        return 0

    lax.fori_loop(0, ROWS_PER_TILE // 80, zero_step, 0)

    wid = c * NS + t
    pltpu.sync_copy(dis_hbm.at[pl.ds(t * ROWS_PER_TILE, ROWS_PER_TILE)],
                    dis_buf)
    plsc.subcore_barrier()

    bufs = (buf0, buf1)
    sems = (sem0, sem1)
    HALF = CHUNKS_B // 2

    def half_loop(h, _):
        pltpu.sync_copy(row3.at[wid, h], row_all)
        pltpu.sync_copy(col3.at[wid, h], col_all)
        pltpu.async_copy(xs_hbm.at[row_all.at[0]], buf0, sem0)

        def pair(g, _):
            for b in range(2):
                j = g * 2 + b
                pltpu.make_async_copy(xs_hbm.at[row_all.at[j]], bufs[b],
                                      sems[b]).wait()

                @pl.when(j >= 1)
                def _():
                    pltpu.make_async_copy(
                        bufs[1 - b], acc_sp.at[col_all.at[j - 1]],
                        ssem).wait()

                @pl.when(j + 1 < HALF)
                def _():
                    pltpu.async_copy(xs_hbm.at[row_all.at[j + 1]],
                                     bufs[1 - b], sems[1 - b])

                pltpu.async_copy(bufs[b], acc_sp.at[col_all.at[j]], ssem,
                                 add=True)
            return 0

        lax.fori_loop(0, HALF // 2, pair, 0)
        pltpu.make_async_copy(bufs[1], acc_sp.at[col_all.at[HALF - 1]],
                              ssem).wait()
        return 0

    lax.fori_loop(0, 2, half_loop, 0)
    plsc.subcore_barrier()

    def out_chunk(k, _):
        r0 = t * ROWS_PER_TILE + k * 80
        pltpu.sync_copy(acc_sp.at[pl.ds(r0, 80)], zsrc)
        _scale_rows(buf0, dis_buf, k * 80, 80, negate=True)
        pltpu.sync_copy(zsrc, accs_out.at[c, pl.ds(r0, 80)])
        return 0

    lax.fori_loop(0, ROWS_PER_TILE // 80, out_chunk, 0)


def _dense_body(x_ref, accs_ref, hid_ref, w0_ref, w1_ref, bc_ref, wih_ref,
                whh_ref, bih_ref, bhh_ref, out_ref):
    tx1 = accs_ref[0] + accs_ref[1]
    h = x_ref[...] @ w0_ref[...] + tx1 @ w1_ref[...] + bc_ref[...]
    h = jnp.maximum(h, 0.0)
    gi = h @ wih_ref[...] + bih_ref[...]
    gh = hid_ref[...] @ whh_ref[...] + bhh_ref[...]
    r = jax.nn.sigmoid(gi[:, :H] + gh[:, :H])
    z = jax.nn.sigmoid(gi[:, H:2 * H] + gh[:, H:2 * H])
    nn_ = jnp.tanh(gi[:, 2 * H:] + r * gh[:, 2 * H:])
    out_ref[...] = (1.0 - z) * nn_ + z * hid_ref[...]


def kernel(x, edge_index, hidden, W0, W1, b_conv, W_ih, W_hh, b_ih, b_hh):
    row = edge_index[0].astype(jnp.int32)
    col = edge_index[1].astype(jnp.int32)
    row3 = row.reshape(NC * NS, 2, CHUNKS_B // 2, CE)
    col3 = col.reshape(NC * NS, 2, CHUNKS_B // 2, CE)
    col3a = col.reshape(NS, CHUNKS_A, CE)

    mesh = plsc.VectorSubcoreMesh(core_axis_name="c", subcore_axis_name="s")

    deg_xs = pl.kernel(
        _deg_xs_body,
        out_type=[
            jax.ShapeDtypeStruct((NP, F), jnp.float32),
            jax.ShapeDtypeStruct((NP,), jnp.float32),
        ],
        mesh=mesh,
        scratch_types=[
            pltpu.VMEM((CHUNKS_A, CE), jnp.int32),
            pltpu.VMEM((CE,), jnp.float32),
            pltpu.VMEM((ROWS_PER_TILE,), jnp.float32),
            pltpu.VMEM((ROWS_PER_TILE,), jnp.float32),
            pltpu.VMEM((XROWS, F), jnp.float32),
            pltpu.VMEM_SHARED((NP,), jnp.float32),
            pltpu.SemaphoreType.DMA,
        ],
    )
    xs, dis = deg_xs(col3a, x)

    edge_pass = pl.kernel(
        _edge_body,
        out_type=jax.ShapeDtypeStruct((NC, NP, F), jnp.float32),
        mesh=mesh,
        scratch_types=[
            pltpu.VMEM((CHUNKS_B // 2, CE), jnp.int32),
            pltpu.VMEM((CHUNKS_B // 2, CE), jnp.int32),
            pltpu.VMEM((CE, F), jnp.float32),
            pltpu.VMEM((CE, F), jnp.float32),
            pltpu.VMEM((ROWS_PER_TILE,), jnp.float32),
            pltpu.VMEM_SHARED((NP, F), jnp.float32),
            pltpu.SemaphoreType.DMA,
            pltpu.SemaphoreType.DMA,
            pltpu.SemaphoreType.DMA,
        ],
    )
    accs = edge_pass(row3, col3, xs, dis)

    R = 1000
    grid = N // R
    new_hidden = pl.pallas_call(
        _dense_body,
        grid=(grid,),
        in_specs=[
            pl.BlockSpec((R, F), lambda i: (i, 0)),
            pl.BlockSpec((NC, R, F), lambda i: (0, i, 0)),
            pl.BlockSpec((R, H), lambda i: (i, 0)),
            pl.BlockSpec((F, H), lambda i: (0, 0)),
            pl.BlockSpec((F, H), lambda i: (0, 0)),
            pl.BlockSpec((1, H), lambda i: (0, 0)),
            pl.BlockSpec((H, 3 * H), lambda i: (0, 0)),
            pl.BlockSpec((H, 3 * H), lambda i: (0, 0)),
            pl.BlockSpec((1, 3 * H), lambda i: (0, 0)),
            pl.BlockSpec((1, 3 * H), lambda i: (0, 0)),
        ],
        out_specs=pl.BlockSpec((R, H), lambda i: (i, 0)),
        out_shape=jax.ShapeDtypeStruct((N, H), jnp.float32),
    )(
        x, accs, hidden, W0, W1, b_conv.reshape(1, H), W_ih.T, W_hh.T,
        b_ih.reshape(1, 3 * H), b_hh.reshape(1, 3 * H),
    )
    return new_hidden


# speedup vs baseline: 1.0480x; 1.0480x over previous
import functools

import jax
import jax.numpy as jnp
from jax import lax
from jax.experimental import pallas as pl
from jax.experimental.pallas import tpu as pltpu
from jax.experimental.pallas import tpu_sc as plsc

N = 10000
E = 320000
F = 128
H = 256
NP = 10240
NC = 2
NS = 16
RPT = NP // NS
CEA = 100
ONESPAD = 112
CHUNKS_A = E // NS // CEA
ASTG = 5
KDEG = 4
CEB = 50
NST = 5
STCH = 40
NBUF = 4
XCH = 40


def _vec_fill(ref, n16, value):
    v = jnp.full((16,), value, dtype=ref.dtype)

    def body(i, _):
        ref[pl.ds(i * 16, 16)] = v
        return 0

    lax.fori_loop(0, n16, body, 0)


def _vec_fill2d(ref, value):
    rows, cols = ref.shape
    v = jnp.full((16,), value, dtype=ref.dtype)

    def body(r, _):
        for u in range(cols // 16):
            ref[r, pl.ds(u * 16, 16)] = v
        return 0

    lax.fori_loop(0, rows, body, 0)


def _scale_rows(buf, dis_ref, dis_base, nrows, negate):
    def group(g, _):
        dvec = dis_ref[pl.ds(dis_base + g * 8, 16)]
        if negate:
            dvec = 0.0 - dvec
        for r8 in range(8):
            s = dvec[r8]
            row = g * 8 + r8
            for u in range(F // 16):
                buf[row, pl.ds(u * 16, 16)] = buf[row, pl.ds(u * 16, 16)] * s
        return 0

    lax.fori_loop(0, nrows // 8, group, 0)


def _fast_rsqrt(d):
    i = lax.bitcast_convert_type(d, jnp.int32)
    i = jnp.int32(0x5F3759DF) - (i >> 1)
    y = lax.bitcast_convert_type(i, jnp.float32)
    for _ in range(3):
        y = y * (1.5 - 0.5 * d * y * y)
    return y


def _sc_body(col3a, row3, col3, x_hbm, accs_out, xs2, deg_idx, ones_buf,
             deg2, dtmp, dis_buf, row_all, col_all, buf0, buf1, buf2, buf3,
             deg_sp, acc_sp, semd, sem0, sem1, sem2, sem3, ssem):
    c = lax.axis_index("c")
    t = lax.axis_index("s")
    wid = c * NS + t

    _vec_fill2d(buf0, 0.0)
    z40 = buf0.at[pl.ds(0, XCH)]

    def zacc(k, _):
        pltpu.sync_copy(z40, acc_sp.at[pl.ds(t * RPT + k * XCH, XCH)])
        return 0

    lax.fori_loop(0, RPT // XCH, zacc, 0)
    _vec_fill(deg2, RPT // 16, 0.0)

    def zdeg(k, _):
        pltpu.sync_copy(deg2, deg_sp.at[pl.ds(k * NP + t * RPT, RPT)])
        return 0

    lax.fori_loop(0, KDEG, zdeg, 0)
    _vec_fill(ones_buf, ONESPAD // 16, 1.0)
    plsc.subcore_barrier()

    ones100 = ones_buf.at[pl.ds(0, CEA)]

    def deg_stage(s, _):
        pltpu.sync_copy(col3a.at[t, pl.ds(s * (CHUNKS_A // ASTG),
                                          CHUNKS_A // ASTG)], deg_idx)

        def batch(bi, _):
            def fire(k, _):
                pltpu.async_copy(ones100,
                                 deg_sp.at[deg_idx.at[bi * KDEG + k]],
                                 semd, add=True)
                return 0

            lax.fori_loop(0, KDEG, fire, 0)

            def drain(k, _):
                pltpu.make_async_copy(
                    ones100, deg_sp.at[deg_idx.at[bi * KDEG + k]],
                    semd).wait()
                return 0

            lax.fori_loop(0, KDEG, drain, 0)
            return 0

        lax.fori_loop(0, CHUNKS_A // ASTG // KDEG, batch, 0)
        return 0

    lax.fori_loop(0, ASTG, deg_stage, 0)
    plsc.subcore_barrier()

    _vec_fill(deg2, RPT // 16, 0.0)

    def acc_k(k, _):
        pltpu.sync_copy(deg_sp.at[pl.ds(k * NP + t * RPT, RPT)], dtmp)

        def add_j(j, _):
            deg2[pl.ds(j * 16, 16)] = (deg2[pl.ds(j * 16, 16)]
                                       + dtmp[pl.ds(j * 16, 16)])
            return 0

        lax.fori_loop(0, RPT // 16, add_j, 0)
        return 0

    lax.fori_loop(0, KDEG, acc_k, 0)

    def dis_step(j, _):
        deg = deg2[pl.ds(j * 16, 16)]
        y = _fast_rsqrt(jnp.maximum(deg, 1.0))
        dis_buf[pl.ds(j * 16, 16)] = jnp.where(deg > 0.0, y, 0.0)
        return 0

    lax.fori_loop(0, RPT // 16, dis_step, 0)

    row0 = t * RPT
    nch = jnp.minimum(N - row0, RPT) // XCH

    def x_chunk(k, _):
        r0 = row0 + k * XCH
        pltpu.sync_copy(x_hbm.at[pl.ds(r0, XCH)], z40)
        _scale_rows(buf0, dis_buf, k * XCH, XCH, negate=False)
        pltpu.sync_copy(z40, xs2.at[pl.ds(c * NP + r0, XCH)])
        return 0

    lax.fori_loop(0, nch, x_chunk, 0)
    plsc.subcore_barrier()

    bufs = (buf0, buf1, buf2, buf3)
    sems = (sem0, sem1, sem2, sem3)

    def stage_loop(st, _):
        pltpu.sync_copy(row3.at[wid, st], row_all)
        pltpu.sync_copy(col3.at[wid, st], col_all)
        for b in range(NBUF - 1):
            pltpu.async_copy(xs2.at[row_all.at[b]], bufs[b], sems[b])

        def quad(q, _):
            for b in range(NBUF):
                j = q * NBUF + b
                pltpu.make_async_copy(xs2.at[row_all.at[j]], bufs[b],
                                      sems[b]).wait()

                @pl.when(j >= 1)
                def _():
                    pltpu.make_async_copy(
                        bufs[(b + NBUF - 1) % NBUF],
                        acc_sp.at[col_all.at[j - 1]], ssem).wait()

                @pl.when(j + NBUF - 1 < STCH)
                def _():
                    pltpu.async_copy(
                        xs2.at[row_all.at[j + NBUF - 1]],
                        bufs[(b + NBUF - 1) % NBUF],
                        sems[(b + NBUF - 1) % NBUF])

                pltpu.async_copy(bufs[b], acc_sp.at[col_all.at[j]], ssem,
                                 add=True)
            return 0

        lax.fori_loop(0, STCH // NBUF, quad, 0)
        pltpu.make_async_copy(bufs[(STCH - 1) % NBUF],
                              acc_sp.at[col_all.at[STCH - 1]], ssem).wait()
        return 0

    lax.fori_loop(0, NST, stage_loop, 0)
    plsc.subcore_barrier()

    def out_chunk(k, _):
        r0 = t * RPT + k * XCH
        pltpu.sync_copy(acc_sp.at[pl.ds(r0, XCH)], z40)
        _scale_rows(buf0, dis_buf, k * XCH, XCH, negate=True)
        pltpu.sync_copy(z40, accs_out.at[c, pl.ds(r0, XCH)])
        return 0

    lax.fori_loop(0, RPT // XCH, out_chunk, 0)


def _dense_body(x_ref, accs_ref, hid_ref, w0_ref, w1_ref, bc_ref, wih_ref,
                whh_ref, bih_ref, bhh_ref, out_ref):
    bf = jnp.bfloat16
    f32 = jnp.float32
    tx1 = (accs_ref[0] + accs_ref[1]).astype(bf)
    h = (jnp.dot(x_ref[...].astype(bf), w0_ref[...],
                 preferred_element_type=f32)
         + jnp.dot(tx1, w1_ref[...], preferred_element_type=f32)
         + bc_ref[...])
    h = jnp.maximum(h, 0.0).astype(bf)
    hid16 = hid_ref[...].astype(bf)
    gi = jnp.dot(h, wih_ref[...], preferred_element_type=f32) + bih_ref[...]
    gh = (jnp.dot(hid16, whh_ref[...], preferred_element_type=f32)
          + bhh_ref[...])
    r = jax.nn.sigmoid(gi[:, :H] + gh[:, :H])
    z = jax.nn.sigmoid(gi[:, H:2 * H] + gh[:, H:2 * H])
    nn_ = jnp.tanh(gi[:, 2 * H:] + r * gh[:, 2 * H:])
    out_ref[...] = (1.0 - z) * nn_ + z * hid_ref[...]


def kernel(x, edge_index, hidden, W0, W1, b_conv, W_ih, W_hh, b_ih, b_hh):
    row = edge_index[0].astype(jnp.int32)
    col = edge_index[1].astype(jnp.int32)
    core_off = (jnp.arange(NC * NS, dtype=jnp.int32) // NS) * NP
    row3 = row.reshape(NC * NS, NST, STCH, CEB) + core_off[:, None, None,
                                                           None]
    col3 = col.reshape(NC * NS, NST, STCH, CEB)
    stripe = (jnp.arange(CHUNKS_A, dtype=jnp.int32) % KDEG) * NP
    col3a = col.reshape(NS, CHUNKS_A, CEA) + stripe[None, :, None]

    mesh = plsc.VectorSubcoreMesh(core_axis_name="c", subcore_axis_name="s")

    sc_pass = pl.kernel(
        _sc_body,
        out_type=[
            jax.ShapeDtypeStruct((NC, NP, F), jnp.float32),
            jax.ShapeDtypeStruct((NC * NP, F), jnp.float32),
        ],
        mesh=mesh,
        scratch_types=[
            pltpu.VMEM((CHUNKS_A // ASTG, CEA), jnp.int32),
            pltpu.VMEM((ONESPAD,), jnp.float32),
            pltpu.VMEM((RPT,), jnp.float32),
            pltpu.VMEM((RPT,), jnp.float32),
            pltpu.VMEM((RPT + 16,), jnp.float32),
            pltpu.VMEM((STCH, CEB), jnp.int32),
            pltpu.VMEM((STCH, CEB), jnp.int32),
            pltpu.VMEM((CEB, F), jnp.float32),
            pltpu.VMEM((CEB, F), jnp.float32),
            pltpu.VMEM((CEB, F), jnp.float32),
            pltpu.VMEM((CEB, F), jnp.float32),
            pltpu.VMEM_SHARED((KDEG * NP,), jnp.float32),
            pltpu.VMEM_SHARED((NP, F), jnp.float32),
            pltpu.SemaphoreType.DMA,
            pltpu.SemaphoreType.DMA,
            pltpu.SemaphoreType.DMA,
            pltpu.SemaphoreType.DMA,
            pltpu.SemaphoreType.DMA,
            pltpu.SemaphoreType.DMA,
        ],
    )
    accs, _ = sc_pass(col3a, row3, col3, x)

    R = 1000
    grid = N // R
    new_hidden = pl.pallas_call(
        _dense_body,
        grid=(grid,),
        in_specs=[
            pl.BlockSpec((R, F), lambda i: (i, 0)),
            pl.BlockSpec((NC, R, F), lambda i: (0, i, 0)),
            pl.BlockSpec((R, H), lambda i: (i, 0)),
            pl.BlockSpec((F, H), lambda i: (0, 0)),
            pl.BlockSpec((F, H), lambda i: (0, 0)),
            pl.BlockSpec((1, H), lambda i: (0, 0)),
            pl.BlockSpec((H, 3 * H), lambda i: (0, 0)),
            pl.BlockSpec((H, 3 * H), lambda i: (0, 0)),
            pl.BlockSpec((1, 3 * H), lambda i: (0, 0)),
            pl.BlockSpec((1, 3 * H), lambda i: (0, 0)),
        ],
        out_specs=pl.BlockSpec((R, H), lambda i: (i, 0)),
        out_shape=jax.ShapeDtypeStruct((N, H), jnp.float32),
    )(
        x, accs, hidden, W0.astype(jnp.bfloat16), W1.astype(jnp.bfloat16),
        b_conv.reshape(1, H), W_ih.T.astype(jnp.bfloat16),
        W_hh.T.astype(jnp.bfloat16),
        b_ih.reshape(1, 3 * H), b_hh.reshape(1, 3 * H),
    )
    return new_hidden


# speedup vs baseline: 1.2543x; 1.1969x over previous
import functools

import jax
import jax.numpy as jnp
from jax import lax
from jax.experimental import pallas as pl
from jax.experimental.pallas import tpu as pltpu
from jax.experimental.pallas import tpu_sc as plsc

N = 10000
E = 320000
F = 128
H = 256
NP = 10240
NC = 2
NS = 16
ROWS_PER_TILE = NP // NS
CEA = 100
ONESPAD = 112
CHUNKS_A = E // NS // CEA
CEB = 50
NST = 5
STCH = 40
NBUF = 5
XROWS = 80
KDEG = 4


def _vec_fill(ref, n16, value):
    v = jnp.full((16,), value, dtype=ref.dtype)

    def body(i, _):
        ref[pl.ds(i * 16, 16)] = v
        return 0

    lax.fori_loop(0, n16, body, 0)


def _vec_fill2d(ref, value):
    rows, cols = ref.shape
    v = jnp.full((16,), value, dtype=ref.dtype)

    def body(r, _):
        for u in range(cols // 16):
            ref[r, pl.ds(u * 16, 16)] = v
        return 0

    lax.fori_loop(0, rows, body, 0)


def _scale_rows(buf, dis_ref, dis_base, nrows, negate):
    def group(g, _):
        dvec = dis_ref[pl.ds(dis_base + g * 16, 16)]
        if negate:
            dvec = 0.0 - dvec
        for r16 in range(16):
            s = dvec[r16]
            row = g * 16 + r16
            for u in range(F // 16):
                buf[row, pl.ds(u * 16, 16)] = buf[row, pl.ds(u * 16, 16)] * s
        return 0

    lax.fori_loop(0, nrows // 16, group, 0)


def _fast_rsqrt(d):
    i = lax.bitcast_convert_type(d, jnp.int32)
    i = jnp.int32(0x5F3759DF) - (i >> 1)
    y = lax.bitcast_convert_type(i, jnp.float32)
    for _ in range(3):
        y = y * (1.5 - 0.5 * d * y * y)
    return y


def _deg_xs_body(col3, x_hbm, xs_out, dis_out, col_all, ones_buf, deg2,
                 dtmp, dis_buf, xbuf, deg_sp, sem):
    c = lax.axis_index("c")
    t = lax.axis_index("s")

    _vec_fill(deg2, ROWS_PER_TILE // 16, 0.0)

    def zero_k(k, _):
        pltpu.sync_copy(deg2, deg_sp.at[pl.ds(k * NP + t * ROWS_PER_TILE,
                                              ROWS_PER_TILE)])
        return 0

    lax.fori_loop(0, KDEG, zero_k, 0)
    _vec_fill(ones_buf, ONESPAD // 16, 1.0)
    pltpu.sync_copy(col3.at[t], col_all)
    plsc.subcore_barrier()

    def deg_batch(bi, _):
        def fire(k, _):
            pltpu.async_copy(ones_buf.at[pl.ds(0, CEA)],
                             deg_sp.at[col_all.at[bi * KDEG + k]],
                             sem, add=True)
            return 0

        lax.fori_loop(0, KDEG, fire, 0)

        def drain(k, _):
            pltpu.make_async_copy(
                ones_buf.at[pl.ds(0, CEA)],
                deg_sp.at[col_all.at[bi * KDEG + k]], sem).wait()
            return 0

        lax.fori_loop(0, KDEG, drain, 0)
        return 0

    lax.fori_loop(0, CHUNKS_A // KDEG, deg_batch, 0)
    plsc.subcore_barrier()

    _vec_fill(deg2, ROWS_PER_TILE // 16, 0.0)

    def acc_k(k, _):
        pltpu.sync_copy(deg_sp.at[pl.ds(k * NP + t * ROWS_PER_TILE,
                                        ROWS_PER_TILE)], dtmp)

        def add_j(j, _):
            deg2[pl.ds(j * 16, 16)] = (deg2[pl.ds(j * 16, 16)]
                                       + dtmp[pl.ds(j * 16, 16)])
            return 0

        lax.fori_loop(0, ROWS_PER_TILE // 16, add_j, 0)
        return 0

    lax.fori_loop(0, KDEG, acc_k, 0)

    def dis_step(j, _):
        deg = deg2[pl.ds(j * 16, 16)]
        y = _fast_rsqrt(jnp.maximum(deg, 1.0))
        dis_buf[pl.ds(j * 16, 16)] = jnp.where(deg > 0.0, y, 0.0)
        return 0

    lax.fori_loop(0, ROWS_PER_TILE // 16, dis_step, 0)

    @pl.when(c == 0)
    def _():
        pltpu.sync_copy(dis_buf, dis_out.at[pl.ds(t * ROWS_PER_TILE,
                                                  ROWS_PER_TILE)])

    row0 = t * ROWS_PER_TILE + c * (ROWS_PER_TILE // 2)
    nch = jnp.minimum(jnp.maximum(N - row0, 0), ROWS_PER_TILE // 2) // XROWS

    def x_chunk(k, _):
        r0 = row0 + k * XROWS
        pltpu.sync_copy(x_hbm.at[pl.ds(r0, XROWS)], xbuf)
        _scale_rows(xbuf, dis_buf, r0 - t * ROWS_PER_TILE, XROWS,
                    negate=False)
        pltpu.sync_copy(xbuf, xs_out.at[pl.ds(r0, XROWS)])
        return 0

    lax.fori_loop(0, nch, x_chunk, 0)


def _edge_body(row3, col3, xs_hbm, dis_hbm, accs_out, row_all, col_all,
               buf0, buf1, buf2, buf3, buf4, dis_buf, acc_sp,
               sem0, sem1, sem2, sem3, sem4, ssem):
    c = lax.axis_index("c")
    t = lax.axis_index("s")

    _vec_fill2d(buf0, 0.0)
    zsrc = buf0.at[pl.ds(0, 80)]

    def zero_step(k, _):
        pltpu.sync_copy(zsrc, acc_sp.at[pl.ds(t * ROWS_PER_TILE + k * 80, 80)])
        return 0

    lax.fori_loop(0, ROWS_PER_TILE // 80, zero_step, 0)

    wid = c * NS + t
    pltpu.sync_copy(dis_hbm.at[pl.ds(t * ROWS_PER_TILE, ROWS_PER_TILE)],
                    dis_buf)
    plsc.subcore_barrier()

    bufs = (buf0, buf1, buf2, buf3, buf4)
    sems = (sem0, sem1, sem2, sem3, sem4)

    def stage_loop(st, _):
        pltpu.sync_copy(row3.at[wid, st], row_all)
        pltpu.sync_copy(col3.at[wid, st], col_all)
        for b in range(NBUF - 1):
            pltpu.async_copy(xs_hbm.at[row_all.at[b]], bufs[b], sems[b])

        def quad(q, _):
            for b in range(NBUF):
                j = q * NBUF + b
                pltpu.make_async_copy(xs_hbm.at[row_all.at[j]], bufs[b],
                                      sems[b]).wait()

                @pl.when(j >= 1)
                def _():
                    pltpu.make_async_copy(
                        bufs[(b + NBUF - 1) % NBUF],
                        acc_sp.at[col_all.at[j - 1]], ssem).wait()

                @pl.when(j + NBUF - 1 < STCH)
                def _():
                    pltpu.async_copy(
                        xs_hbm.at[row_all.at[j + NBUF - 1]],
                        bufs[(b + NBUF - 1) % NBUF],
                        sems[(b + NBUF - 1) % NBUF])

                pltpu.async_copy(bufs[b], acc_sp.at[col_all.at[j]], ssem,
                                 add=True)
            return 0

        lax.fori_loop(0, STCH // NBUF, quad, 0)
        pltpu.make_async_copy(bufs[(STCH - 1) % NBUF],
                              acc_sp.at[col_all.at[STCH - 1]], ssem).wait()
        return 0

    lax.fori_loop(0, NST, stage_loop, 0)
    plsc.subcore_barrier()

    def out_chunk(k, _):
        r0 = t * ROWS_PER_TILE + k * 80
        pltpu.sync_copy(acc_sp.at[pl.ds(r0, 80)], zsrc)
        _scale_rows(buf0, dis_buf, k * 80, 80, negate=True)
        pltpu.sync_copy(zsrc, accs_out.at[c, pl.ds(r0, 80)])
        return 0

    lax.fori_loop(0, ROWS_PER_TILE // 80, out_chunk, 0)


def _dense_body(x_ref, accs_ref, hid_ref, w0_ref, w1_ref, bc_ref, wih_ref,
                whh_ref, bih_ref, bhh_ref, out_ref):
    bf = jnp.bfloat16
    f32 = jnp.float32
    tx1 = (accs_ref[0] + accs_ref[1]).astype(bf)
    h = (jnp.dot(x_ref[...].astype(bf), w0_ref[...],
                 preferred_element_type=f32)
         + jnp.dot(tx1, w1_ref[...], preferred_element_type=f32)
         + bc_ref[...])
    h = jnp.maximum(h, 0.0).astype(bf)
    hid16 = hid_ref[...].astype(bf)
    gi = jnp.dot(h, wih_ref[...], preferred_element_type=f32) + bih_ref[...]
    gh = (jnp.dot(hid16, whh_ref[...], preferred_element_type=f32)
          + bhh_ref[...])
    r = jax.nn.sigmoid(gi[:, :H] + gh[:, :H])
    z = jax.nn.sigmoid(gi[:, H:2 * H] + gh[:, H:2 * H])
    nn_ = jnp.tanh(gi[:, 2 * H:] + r * gh[:, 2 * H:])
    out_ref[...] = (1.0 - z) * nn_ + z * hid_ref[...]


def kernel(x, edge_index, hidden, W0, W1, b_conv, W_ih, W_hh, b_ih, b_hh):
    row = edge_index[0].astype(jnp.int32)
    col = edge_index[1].astype(jnp.int32)
    row3 = row.reshape(NC * NS, NST, STCH, CEB)
    col3 = col.reshape(NC * NS, NST, STCH, CEB)
    stripe = (jnp.arange(CHUNKS_A, dtype=jnp.int32) % KDEG) * NP
    col3a = col.reshape(NS, CHUNKS_A, CEA) + stripe[None, :, None]

    mesh = plsc.VectorSubcoreMesh(core_axis_name="c", subcore_axis_name="s")

    deg_xs = pl.kernel(
        _deg_xs_body,
        out_type=[
            jax.ShapeDtypeStruct((NP, F), jnp.float32),
            jax.ShapeDtypeStruct((NP,), jnp.float32),
        ],
        mesh=mesh,
        scratch_types=[
            pltpu.VMEM((CHUNKS_A, CEA), jnp.int32),
            pltpu.VMEM((ONESPAD,), jnp.float32),
            pltpu.VMEM((ROWS_PER_TILE,), jnp.float32),
            pltpu.VMEM((ROWS_PER_TILE,), jnp.float32),
            pltpu.VMEM((ROWS_PER_TILE,), jnp.float32),
            pltpu.VMEM((XROWS, F), jnp.float32),
            pltpu.VMEM_SHARED((KDEG * NP,), jnp.float32),
            pltpu.SemaphoreType.DMA,
        ],
    )
    xs, dis = deg_xs(col3a, x)

    edge_pass = pl.kernel(
        _edge_body,
        out_type=jax.ShapeDtypeStruct((NC, NP, F), jnp.float32),
        mesh=mesh,
        scratch_types=[
            pltpu.VMEM((STCH, CEB), jnp.int32),
            pltpu.VMEM((STCH, CEB), jnp.int32),
            pltpu.VMEM((CEB, F), jnp.float32),
            pltpu.VMEM((CEB, F), jnp.float32),
            pltpu.VMEM((CEB, F), jnp.float32),
            pltpu.VMEM((CEB, F), jnp.float32),
            pltpu.VMEM((CEB, F), jnp.float32),
            pltpu.VMEM((ROWS_PER_TILE,), jnp.float32),
            pltpu.VMEM_SHARED((NP, F), jnp.float32),
            pltpu.SemaphoreType.DMA,
            pltpu.SemaphoreType.DMA,
            pltpu.SemaphoreType.DMA,
            pltpu.SemaphoreType.DMA,
            pltpu.SemaphoreType.DMA,
            pltpu.SemaphoreType.DMA,
        ],
    )
    accs = edge_pass(row3, col3, xs, dis)

    R = 1000
    grid = N // R
    new_hidden = pl.pallas_call(
        _dense_body,
        grid=(grid,),
        in_specs=[
            pl.BlockSpec((R, F), lambda i: (i, 0)),
            pl.BlockSpec((NC, R, F), lambda i: (0, i, 0)),
            pl.BlockSpec((R, H), lambda i: (i, 0)),
            pl.BlockSpec((F, H), lambda i: (0, 0)),
            pl.BlockSpec((F, H), lambda i: (0, 0)),
            pl.BlockSpec((1, H), lambda i: (0, 0)),
            pl.BlockSpec((H, 3 * H), lambda i: (0, 0)),
            pl.BlockSpec((H, 3 * H), lambda i: (0, 0)),
            pl.BlockSpec((1, 3 * H), lambda i: (0, 0)),
            pl.BlockSpec((1, 3 * H), lambda i: (0, 0)),
        ],
        out_specs=pl.BlockSpec((R, H), lambda i: (i, 0)),
        out_shape=jax.ShapeDtypeStruct((N, H), jnp.float32),
    )(
        x, accs, hidden, W0.astype(jnp.bfloat16), W1.astype(jnp.bfloat16),
        b_conv.reshape(1, H), W_ih.T.astype(jnp.bfloat16),
        W_hh.T.astype(jnp.bfloat16),
        b_ih.reshape(1, 3 * H), b_hh.reshape(1, 3 * H),
    )
    return new_hidden
